# baseline (speedup 1.0000x reference)
import functools

import jax
import jax.numpy as jnp
from jax import lax
from jax.experimental import pallas as pl
from jax.experimental.pallas import tpu as pltpu
from jax.experimental.pallas import tpu_sc as plsc

B = 16384
D = 128
NC = 2
NS = 16
L = 16
NW = NC * NS
BPW = B // NW
C = 128
NCHUNK = BPW // C


def _sc_body(x, ent, rel, wb, conf_out, part_out,
             x_v, idxh_v, idxr_v, idxt_v, h0, r0, t0, h1, r1, t1, conf_v,
             wb_v, part_v, sem0, sem1):
    cid = lax.axis_index("c")
    sid = lax.axis_index("s")
    wid = sid * NC + cid
    base = wid * BPW

    pltpu.sync_copy(x.at[pl.ds(base * 3, BPW * 3)], x_v)
    pltpu.sync_copy(wb, wb_v.at[pl.ds(0, 2)])
    lane = lax.iota(jnp.int32, L)
    zero = jnp.zeros((L,), jnp.float32)
    wbv = wb_v[...]
    w = jnp.full((L,), wbv[0])
    b = jnp.full((L,), wbv[1])

    @plsc.parallel_loop(0, BPW // L, unroll=2)
    def deint(i):
        rows3 = (lane + i * L) * 3
        for col, dst in ((0, idxh_v), (1, idxr_v), (2, idxt_v)):
            dst[pl.ds(i * L, L)] = plsc.load_gather(x_v, [rows3 + col])

    bufs = ((h0, r0, t0, sem0), (h1, r1, t1, sem1))

    def fire(cc, b_):
        hb, rb, tb, sem = bufs[b_]
        pltpu.async_copy(ent.at[idxh_v.at[pl.ds(cc * C, C)]], hb, sem)
        pltpu.async_copy(rel.at[idxr_v.at[pl.ds(cc * C, C)]], rb, sem)
        pltpu.async_copy(ent.at[idxt_v.at[pl.ds(cc * C, C)]], tb, sem)

    def drain(b_):
        hb, rb, tb, sem = bufs[b_]
        pltpu.make_async_copy(ent.at[idxh_v.at[pl.ds(0, C)]], hb, sem).wait()
        pltpu.make_async_copy(rel.at[idxr_v.at[pl.ds(0, C)]], rb, sem).wait()
        pltpu.make_async_copy(ent.at[idxt_v.at[pl.ds(0, C)]], tb, sem).wait()

    def compute_chunk(cc, b_, sq4):
        hb, rb, tb, _ = bufs[b_]

        def gbody(g, sq4):
            @plsc.parallel_loop(0, L, carry=(*sq4, zero), unroll=2)
            def jloop(j, carry):
                s0, s1, s2, s3, pv = carry
                row = g * L + j
                hs = [hb[row, pl.ds(L * k, L)] for k in range(8)]
                ts = [tb[row, pl.ds(L * k, L)] for k in range(8)]
                rs = [rb[row, pl.ds(L * k, L)] for k in range(8)]
                ms = [(hs[k] * ts[k]) * rs[k] for k in range(8)]
                p_sum = jnp.sum(((ms[0] + ms[1]) + (ms[2] + ms[3])) +
                                ((ms[4] + ms[5]) + (ms[6] + ms[7])))
                pv = jnp.where(lane == j, p_sum, pv)
                accs = [s0, s1, s2, s3]
                vs = hs + ts + rs
                for k in range(24):
                    accs[k % 4] = accs[k % 4] + vs[k] * vs[k]
                return (*accs, pv)

            s0, s1, s2, s3, pv = jloop
            z = pv * w + b
            conf_v[pl.ds(cc * C + g * L, L)] = 1.0 / (1.0 + jnp.exp(-z))
            return (s0, s1, s2, s3)

        return lax.fori_loop(0, C // L, gbody, sq4)

    sq4 = (zero, zero, zero, zero)
    fire(0, 0)
    for cc in range(NCHUNK):
        drain(cc % 2)
        if cc + 1 < NCHUNK:
            fire(cc + 1, (cc + 1) % 2)
        sq4 = compute_chunk(cc, cc % 2, sq4)

    part_v[...] = ((sq4[0] + sq4[1]) + (sq4[2] + sq4[3]))
    pltpu.sync_copy(conf_v, conf_out.at[pl.ds(base, BPW)])
    pltpu.sync_copy(part_v, part_out.at[wid])


_sc_call = functools.partial(
    pl.kernel,
    out_type=[
        jax.ShapeDtypeStruct((B,), jnp.float32),
        jax.ShapeDtypeStruct((NW, L), jnp.float32),
    ],
    mesh=plsc.VectorSubcoreMesh(core_axis_name="c", subcore_axis_name="s"),
    compiler_params=pltpu.CompilerParams(needs_layout_passes=False),
    scratch_types=[
        pltpu.VMEM((BPW * 3,), jnp.int32),
        pltpu.VMEM((BPW,), jnp.int32),
        pltpu.VMEM((BPW,), jnp.int32),
        pltpu.VMEM((BPW,), jnp.int32),
        pltpu.VMEM((C, D), jnp.float32),
        pltpu.VMEM((C, D), jnp.float32),
        pltpu.VMEM((C, D), jnp.float32),
        pltpu.VMEM((C, D), jnp.float32),
        pltpu.VMEM((C, D), jnp.float32),
        pltpu.VMEM((C, D), jnp.float32),
        pltpu.VMEM((BPW,), jnp.float32),
        pltpu.VMEM((L,), jnp.float32),
        pltpu.VMEM((L,), jnp.float32),
        pltpu.SemaphoreType.DMA,
        pltpu.SemaphoreType.DMA,
    ],
)(_sc_body)


def _finish_body(p_ref, o_ref):
    o_ref[0, 0] = jnp.sum(p_ref[...]) * (1.0 / (float(B) * float(B) * float(D)))


_finish = pl.pallas_call(
    _finish_body,
    out_shape=jax.ShapeDtypeStruct((1, 1), jnp.float32),
    out_specs=pl.BlockSpec(memory_space=pltpu.SMEM),
)


def kernel(x, entityEmbed, relationEmbed, lin_w, lin_b):
    x = x.astype(jnp.int32).reshape(-1)
    wb = jnp.concatenate([lin_w[0].astype(jnp.float32),
                          lin_b.astype(jnp.float32)])
    conf, part = _sc_call(x, entityEmbed, relationEmbed, wb)
    r_score = _finish(part)[0, 0]
    return conf, r_score


# speedup vs baseline: 1.2417x; 1.2417x over previous
import functools

import jax
import jax.numpy as jnp
from jax import lax
from jax.experimental import pallas as pl
from jax.experimental.pallas import tpu as pltpu
from jax.experimental.pallas import tpu_sc as plsc

B = 16384
D = 128
NC = 2
NS = 16
L = 16
NW = NC * NS
BPW = B // NW
C = 128
NCHUNK = BPW // C


def _sc_body(hidx, ridx, tidx, ent, rel, w16, b16, conf_out, part_out,
             idxh_v, idxr_v, idxt_v, h0, r0, t0, h1, r1, t1, conf_v,
             w_v, b_v, part_v, sem0, sem1):
    cid = lax.axis_index("c")
    sid = lax.axis_index("s")
    wid = sid * NC + cid
    base = wid * BPW

    pltpu.sync_copy(hidx.at[pl.ds(base, BPW)], idxh_v)
    pltpu.sync_copy(ridx.at[pl.ds(base, BPW)], idxr_v)
    pltpu.sync_copy(tidx.at[pl.ds(base, BPW)], idxt_v)
    pltpu.sync_copy(w16, w_v)
    pltpu.sync_copy(b16, b_v)
    lane = lax.iota(jnp.int32, L)
    zero = jnp.zeros((L,), jnp.float32)
    w = w_v[...]
    b = b_v[...]

    bufs = ((h0, r0, t0, sem0), (h1, r1, t1, sem1))

    def fire(cc, b_):
        hb, rb, tb, sem = bufs[b_]
        pltpu.async_copy(ent.at[idxh_v.at[pl.ds(cc * C, C)]], hb, sem)
        pltpu.async_copy(rel.at[idxr_v.at[pl.ds(cc * C, C)]], rb, sem)
        pltpu.async_copy(ent.at[idxt_v.at[pl.ds(cc * C, C)]], tb, sem)

    def drain(b_):
        hb, rb, tb, sem = bufs[b_]
        pltpu.make_async_copy(ent.at[idxh_v.at[pl.ds(0, C)]], hb, sem).wait()
        pltpu.make_async_copy(rel.at[idxr_v.at[pl.ds(0, C)]], rb, sem).wait()
        pltpu.make_async_copy(ent.at[idxt_v.at[pl.ds(0, C)]], tb, sem).wait()

    def compute_chunk(cc, b_, sq4):
        hb, rb, tb, _ = bufs[b_]

        def gbody(g, sq4):
            @plsc.parallel_loop(0, L, carry=(*sq4, zero), unroll=2)
            def jloop(j, carry):
                s0, s1, s2, s3, pv = carry
                row = g * L + j
                hs = [hb[row, pl.ds(L * k, L)] for k in range(8)]
                ts = [tb[row, pl.ds(L * k, L)] for k in range(8)]
                rs = [rb[row, pl.ds(L * k, L)] for k in range(8)]
                ms = [(hs[k] * ts[k]) * rs[k] for k in range(8)]
                p_sum = jnp.sum(((ms[0] + ms[1]) + (ms[2] + ms[3])) +
                                ((ms[4] + ms[5]) + (ms[6] + ms[7])))
                pv = jnp.where(lane == j, p_sum, pv)
                accs = [s0, s1, s2, s3]
                vs = hs + ts + rs
                for k in range(24):
                    accs[k % 4] = accs[k % 4] + vs[k] * vs[k]
                return (*accs, pv)

            s0, s1, s2, s3, pv = jloop
            z = pv * w + b
            conf_v[pl.ds(cc * C + g * L, L)] = 1.0 / (1.0 + jnp.exp(-z))
            return (s0, s1, s2, s3)

        return lax.fori_loop(0, C // L, gbody, sq4)

    sq4 = (zero, zero, zero, zero)
    fire(0, 0)
    for cc in range(NCHUNK):
        drain(cc % 2)
        if cc + 1 < NCHUNK:
            fire(cc + 1, (cc + 1) % 2)
        sq4 = compute_chunk(cc, cc % 2, sq4)

    part_v[...] = ((sq4[0] + sq4[1]) + (sq4[2] + sq4[3]))
    pltpu.sync_copy(conf_v, conf_out.at[pl.ds(base, BPW)])
    pltpu.sync_copy(part_v, part_out.at[wid])


_sc_call = functools.partial(
    pl.kernel,
    out_type=[
        jax.ShapeDtypeStruct((B,), jnp.float32),
        jax.ShapeDtypeStruct((NW, L), jnp.float32),
    ],
    mesh=plsc.VectorSubcoreMesh(core_axis_name="c", subcore_axis_name="s"),
    compiler_params=pltpu.CompilerParams(needs_layout_passes=False),
    scratch_types=[
        pltpu.VMEM((BPW,), jnp.int32),
        pltpu.VMEM((BPW,), jnp.int32),
        pltpu.VMEM((BPW,), jnp.int32),
        pltpu.VMEM((C, D), jnp.float32),
        pltpu.VMEM((C, D), jnp.float32),
        pltpu.VMEM((C, D), jnp.float32),
        pltpu.VMEM((C, D), jnp.float32),
        pltpu.VMEM((C, D), jnp.float32),
        pltpu.VMEM((C, D), jnp.float32),
        pltpu.VMEM((BPW,), jnp.float32),
        pltpu.VMEM((L,), jnp.float32),
        pltpu.VMEM((L,), jnp.float32),
        pltpu.VMEM((L,), jnp.float32),
        pltpu.SemaphoreType.DMA,
        pltpu.SemaphoreType.DMA,
    ],
)(_sc_body)


def _finish_body(p_ref, o_ref):
    o_ref[0, 0] = jnp.sum(p_ref[...]) * (1.0 / (float(B) * float(B) * float(D)))


_finish = pl.pallas_call(
    _finish_body,
    out_shape=jax.ShapeDtypeStruct((1, 1), jnp.float32),
    out_specs=pl.BlockSpec(memory_space=pltpu.SMEM),
)


def kernel(x, entityEmbed, relationEmbed, lin_w, lin_b):
    x = x.astype(jnp.int32)
    w16 = jnp.full((L,), lin_w[0, 0], jnp.float32)
    b16 = jnp.full((L,), lin_b[0], jnp.float32)
    conf, part = _sc_call(x[:, 0], x[:, 1], x[:, 2],
                          entityEmbed, relationEmbed, w16, b16)
    r_score = _finish(part)[0, 0]
    return conf, r_score
